# baseline (speedup 1.0000x reference)
import functools

import jax
import jax.numpy as jnp
from jax import lax
from jax.experimental import pallas as pl
from jax.experimental.pallas import tpu as pltpu
from jax.experimental.pallas import tpu_sc as plsc

D = H = W = 128
C = 16
N = 524288
V = D * H * W

NC = 2
NS = 16
NW = NC * NS
NPW = N // NW
B = 256
NBLK = NPW // B
G = B // 16
NIDX = 8 * B
ILEN = 128
NGB = N // B
IROWS = NIDX // ILEN
NSTREAM = NIDX // ILEN

GR = V // W
RPW = GR // NW
RCH = 16
QCH = RCH * W
NCH = RPW // RCH

_mesh = plsc.VectorSubcoreMesh(core_axis_name="c", subcore_axis_name="s")


@functools.partial(
    pl.kernel,
    mesh=_mesh,
    compiler_params=pltpu.CompilerParams(use_tc_tiling_on_sc=False),
    out_type=jax.ShapeDtypeStruct((V, C), jnp.float32),
    scratch_types=[
        pltpu.VMEM((C, RCH, W), jnp.float32),
        pltpu.VMEM((QCH, C), jnp.float32),
        pltpu.SemaphoreType.DMA,
    ],
)
def _transpose_sc(feat_hbm, table_hbm, chan_v, tout_v, sem):
    wid = lax.axis_index("s") * NC + lax.axis_index("c")
    lanes = lax.iota(jnp.int32, 16)
    perms = {d: lanes ^ d for d in (1, 2, 4, 8)}
    masks = {}
    for d in (1, 2, 4, 8):
        for bit in (0, d):
            masks[(d, bit)] = (lanes & d) == bit

    def chunk_body(ch, carry):
        r0 = wid * RPW + ch * RCH
        handles = [
            pltpu.async_copy(feat_hbm.at[c, pl.ds(r0, RCH)], chan_v.at[c], sem)
            for c in range(C)
        ]
        for h in handles:
            h.wait()

        def row_body(r, c2):
            for xb in range(8):
                x0 = xb * 16
                regs = [chan_v[c, r, pl.ds(x0, 16)] for c in range(C)]
                for d in (1, 2, 4, 8):
                    regs = [
                        jnp.where(
                            masks[(d, rr & d)],
                            regs[rr],
                            jnp.take(regs[rr ^ d], perms[d]),
                        )
                        for rr in range(C)
                    ]
                lp0 = r * W + x0
                for j in range(16):
                    tout_v[lp0 + j] = regs[j]
            return c2

        lax.fori_loop(0, RCH, row_body, 0, unroll=False)
        pltpu.sync_copy(tout_v, table_hbm.at[pl.ds(r0 * W, QCH)])
        return carry

    lax.fori_loop(0, NCH, chunk_body, 0, unroll=False)


@functools.partial(
    pl.kernel,
    mesh=_mesh,
    compiler_params=pltpu.CompilerParams(use_tc_tiling_on_sc=False),
    out_type=jax.ShapeDtypeStruct((N // 8, 8 * C), jnp.float32),
    scratch_types=[
        pltpu.VMEM((NIDX,), jnp.int32),
        pltpu.VMEM((NIDX,), jnp.int32),
        pltpu.VMEM((NIDX,), jnp.float32),
        pltpu.VMEM((NIDX,), jnp.float32),
        pltpu.VMEM((NIDX, C), jnp.float32),
        pltpu.VMEM((NIDX, C), jnp.float32),
        pltpu.VMEM((B // 8, 8 * C), jnp.float32),
        pltpu.VMEM((B // 8, 8 * C), jnp.float32),
        pltpu.SemaphoreType.DMA,
        pltpu.SemaphoreType.DMA,
        pltpu.SemaphoreType.DMA,
        pltpu.SemaphoreType.DMA,
        pltpu.SemaphoreType.DMA,
        pltpu.SemaphoreType.DMA,
    ],
)
def _sample_sc(idx_hbm, w_hbm, table_hbm, out_hbm,
               idx0, idx1, w0, w1, rows0, rows1, out0, out1,
               sem0, sem1, psem0, psem1, osem0, osem1):
    wid = lax.axis_index("s") * NC + lax.axis_index("c")
    idxb = [idx0, idx1]
    wb = [w0, w1]
    rowsb = [rows0, rows1]
    outb = [out0, out1]
    semb = [sem0, sem1]
    psemb = [psem0, psem1]
    osemb = [osem0, osem1]

    def fire_prefetch(blk, buf):
        bc = jnp.minimum(blk, NBLK - 1)
        r0 = (wid * NBLK + bc) * NIDX
        pltpu.async_copy(idx_hbm.at[pl.ds(r0, NIDX)], idxb[buf], psemb[buf])
        pltpu.async_copy(w_hbm.at[pl.ds(r0, NIDX)], wb[buf], psemb[buf])

    def wait_prefetch(buf):
        pltpu.make_async_copy(idx_hbm.at[pl.ds(0, NIDX)], idxb[buf],
                              psemb[buf]).wait()
        pltpu.make_async_copy(w_hbm.at[pl.ds(0, NIDX)], wb[buf],
                              psemb[buf]).wait()

    def fire_gathers(buf):
        for j in range(NSTREAM):
            pltpu.async_copy(
                table_hbm.at[idxb[buf].at[pl.ds(j * ILEN, ILEN)]],
                rowsb[buf].at[pl.ds(j * ILEN, ILEN)],
                semb[buf],
            )

    def wait_gathers(buf):
        for j in range(NSTREAM):
            pltpu.make_async_copy(
                table_hbm.at[idxb[buf].at[pl.ds(j * ILEN, ILEN)]],
                rowsb[buf].at[pl.ds(j * ILEN, ILEN)],
                semb[buf],
            ).wait()

    def accum(buf):
        w_v = wb[buf]
        rows_v = rowsb[buf]
        out_v = outb[buf]

        def acc_body(g, c2):
            b0 = g * 16
            wks = [w_v[pl.ds(k * B + b0, 16)] for k in range(8)]
            for j in range(16):
                lane_j = jnp.full((16,), j, jnp.int32)
                acc = None
                for k in range(8):
                    row = rows_v[k * B + b0 + j]
                    wjk = jnp.take(wks[k], lane_j)
                    term = row * wjk
                    acc = term if acc is None else acc + term
                out_v[2 * g + j // 8, pl.ds((j % 8) * C, 16)] = acc
            return c2

        lax.fori_loop(0, G, acc_body, 0, unroll=False)

    def fire_out(blk, buf):
        base = (wid * NPW + blk * B) // 8
        pltpu.async_copy(outb[buf], out_hbm.at[pl.ds(base, B // 8)],
                         osemb[buf])

    def wait_out(buf):
        pltpu.make_async_copy(outb[buf], out_hbm.at[pl.ds(0, B // 8)],
                              osemb[buf]).wait()

    def sub_block(i, buf):
        nbuf = 1 - buf
        wait_prefetch(nbuf)
        fire_gathers(nbuf)
        wait_gathers(buf)
        fire_prefetch(i + 2, buf)

        @pl.when(i >= 2)
        def _():
            wait_out(buf)

        accum(buf)
        fire_out(i, buf)

    fire_prefetch(jnp.int32(0), 0)
    fire_prefetch(jnp.int32(1), 1)
    wait_prefetch(0)
    fire_gathers(0)

    def body(d, carry):
        sub_block(2 * d, 0)
        sub_block(2 * d + 1, 1)
        return carry

    lax.fori_loop(0, NBLK // 2, body, 0, unroll=False)

    wait_gathers(0)
    wait_prefetch(1)
    wait_out(0)
    wait_out(1)


@jax.jit
def kernel(coords, features):
    feat3 = features.reshape(C, GR, W)
    table = _transpose_sc(feat3)

    gg = coords * 2.0 - 1.0
    u = (gg + 1.0) * 0.5 * (W - 1)
    u = jnp.minimum(jnp.maximum(u, 0.0), float(W - 1))
    u0 = u.astype(jnp.int32)
    wu = u - u0.astype(jnp.float32)
    u1 = jnp.minimum(u0 + 1, W - 1)
    x0, y0, z0 = u0[:, 0], u0[:, 1], u0[:, 2]
    x1, y1, z1 = u1[:, 0], u1[:, 1], u1[:, 2]
    wx, wy, wz = wu[:, 0], wu[:, 1], wu[:, 2]
    wx0, wy0, wz0 = 1.0 - wx, 1.0 - wy, 1.0 - wz
    idx_list = []
    w_list = []
    for dz in (0, 1):
        zi = z1 if dz else z0
        wzs = wz if dz else wz0
        for dy in (0, 1):
            yi = y1 if dy else y0
            wys = wy if dy else wy0
            zy = (zi * H + yi) * W
            wzy = wzs * wys
            for dx in (0, 1):
                xi = x1 if dx else x0
                wxs = wx if dx else wx0
                idx_list.append(zy + xi)
                w_list.append(wzy * wxs)
    idxs = jnp.stack(idx_list)
    ws = jnp.stack(w_list)
    idx_t = idxs.reshape(8, NGB, B).transpose(1, 0, 2).reshape(NGB * NIDX)
    w_t = ws.reshape(8, NGB, B).transpose(1, 0, 2).reshape(NGB * NIDX)

    out = _sample_sc(idx_t, w_t, table)
    return out.reshape(N, 1, C)


# speedup vs baseline: 1.2334x; 1.2334x over previous
import functools

import jax
import jax.numpy as jnp
from jax import lax
from jax.experimental import pallas as pl
from jax.experimental.pallas import tpu as pltpu
from jax.experimental.pallas import tpu_sc as plsc

D = H = W = 128
C = 16
N = 524288
V = D * H * W

NC = 2
NS = 16
NW = NC * NS
NPW = N // NW
B = 256
NBLK = NPW // B
G = B // 16
NIDX = 8 * B
ILEN = 128
NGB = N // B
IROWS = NIDX // ILEN
NSTREAM = NIDX // ILEN

GR = V // W
RPW = GR // NW
RCH = 16
QCH = RCH * W
NCH = RPW // RCH

_mesh = plsc.VectorSubcoreMesh(core_axis_name="c", subcore_axis_name="s")


@functools.partial(
    pl.kernel,
    mesh=_mesh,
    compiler_params=pltpu.CompilerParams(use_tc_tiling_on_sc=False),
    out_type=jax.ShapeDtypeStruct((V, C), jnp.float32),
    scratch_types=[
        pltpu.VMEM((C, RCH, W), jnp.float32),
        pltpu.VMEM((QCH, C), jnp.float32),
        pltpu.SemaphoreType.DMA,
    ],
)
def _transpose_sc(feat_hbm, table_hbm, chan_v, tout_v, sem):
    wid = lax.axis_index("s") * NC + lax.axis_index("c")
    lanes = lax.iota(jnp.int32, 16)
    perms = {d: lanes ^ d for d in (1, 2, 4, 8)}
    masks = {}
    for d in (1, 2, 4, 8):
        for bit in (0, d):
            masks[(d, bit)] = (lanes & d) == bit

    def chunk_body(ch, carry):
        r0 = wid * RPW + ch * RCH
        handles = [
            pltpu.async_copy(feat_hbm.at[c, pl.ds(r0, RCH)], chan_v.at[c], sem)
            for c in range(C)
        ]
        for h in handles:
            h.wait()

        def row_body(r, c2):
            for xb in range(8):
                x0 = xb * 16
                regs = [chan_v[c, r, pl.ds(x0, 16)] for c in range(C)]
                for d in (1, 2, 4, 8):
                    regs = [
                        jnp.where(
                            masks[(d, rr & d)],
                            regs[rr],
                            jnp.take(regs[rr ^ d], perms[d]),
                        )
                        for rr in range(C)
                    ]
                lp0 = r * W + x0
                for j in range(16):
                    tout_v[lp0 + j] = regs[j]
            return c2

        lax.fori_loop(0, RCH, row_body, 0, unroll=False)
        pltpu.sync_copy(tout_v, table_hbm.at[pl.ds(r0 * W, QCH)])
        return carry

    lax.fori_loop(0, NCH, chunk_body, 0, unroll=False)


@functools.partial(
    pl.kernel,
    mesh=_mesh,
    compiler_params=pltpu.CompilerParams(use_tc_tiling_on_sc=False),
    out_type=jax.ShapeDtypeStruct((N // 8, 8 * C), jnp.float32),
    scratch_types=[
        pltpu.VMEM((B,), jnp.float32),
        pltpu.VMEM((B,), jnp.float32),
        pltpu.VMEM((B,), jnp.float32),
        pltpu.VMEM((B,), jnp.float32),
        pltpu.VMEM((B,), jnp.float32),
        pltpu.VMEM((B,), jnp.float32),
        pltpu.VMEM((NIDX,), jnp.int32),
        pltpu.VMEM((NIDX,), jnp.int32),
        pltpu.VMEM((NIDX,), jnp.float32),
        pltpu.VMEM((NIDX,), jnp.float32),
        pltpu.VMEM((NIDX, C), jnp.float32),
        pltpu.VMEM((NIDX, C), jnp.float32),
        pltpu.VMEM((B // 8, 8 * C), jnp.float32),
        pltpu.VMEM((B // 8, 8 * C), jnp.float32),
        pltpu.SemaphoreType.DMA,
        pltpu.SemaphoreType.DMA,
        pltpu.SemaphoreType.DMA,
        pltpu.SemaphoreType.DMA,
        pltpu.SemaphoreType.DMA,
        pltpu.SemaphoreType.DMA,
    ],
)
def _sample_sc(xs_hbm, ys_hbm, zs_hbm, table_hbm, out_hbm,
               xv0, yv0, zv0, xv1, yv1, zv1,
               idx0, idx1, w0, w1, rows0, rows1, out0, out1,
               sem0, sem1, csem0, csem1, osem0, osem1):
    wid = lax.axis_index("s") * NC + lax.axis_index("c")
    lanes = lax.iota(jnp.int32, 16)
    cv = [(xv0, yv0, zv0), (xv1, yv1, zv1)]
    idxb = [idx0, idx1]
    wb = [w0, w1]
    rowsb = [rows0, rows1]
    outb = [out0, out1]
    semb = [sem0, sem1]
    csemb = [csem0, csem1]
    osemb = [osem0, osem1]

    def fire_coords(blk, buf):
        bc = jnp.minimum(blk, NBLK - 1)
        base = wid * NPW + bc * B
        pltpu.async_copy(xs_hbm.at[pl.ds(base, B)], cv[buf][0], csemb[buf])
        pltpu.async_copy(ys_hbm.at[pl.ds(base, B)], cv[buf][1], csemb[buf])
        pltpu.async_copy(zs_hbm.at[pl.ds(base, B)], cv[buf][2], csemb[buf])

    def wait_coords(buf):
        for r in cv[buf]:
            pltpu.make_async_copy(xs_hbm.at[pl.ds(0, B)], r, csemb[buf]).wait()

    def idx_weights(buf):
        xv, yv, zv = cv[buf]
        idx_v = idxb[buf]
        w_v = wb[buf]

        def grp_body(g, c2):
            b0 = g * 16
            cx = xv[pl.ds(b0, 16)]
            cy = yv[pl.ds(b0, 16)]
            cz = zv[pl.ds(b0, 16)]

            def axis(cu, ext):
                gg = cu * 2.0 - 1.0
                u = (gg + 1.0) * 0.5 * (ext - 1)
                u = jnp.minimum(jnp.maximum(u, 0.0), float(ext - 1))
                u0 = u.astype(jnp.int32)
                wu = u - u0.astype(jnp.float32)
                u1 = jnp.minimum(u0 + 1, ext - 1)
                return u0, u1, wu

            x0, x1, wx = axis(cx, W)
            y0, y1, wy = axis(cy, H)
            z0, z1, wz = axis(cz, D)
            wx0 = 1.0 - wx
            wy0 = 1.0 - wy
            wz0 = 1.0 - wz
            k = 0
            for dz in (0, 1):
                zi = z1 if dz else z0
                wzs = wz if dz else wz0
                for dy in (0, 1):
                    yi = y1 if dy else y0
                    wys = wy if dy else wy0
                    zy = (zi * H + yi) * W
                    wzy = wzs * wys
                    for dx in (0, 1):
                        xi = x1 if dx else x0
                        wxs = wx if dx else wx0
                        idx_v[pl.ds(k * B + b0, 16)] = zy + xi
                        w_v[pl.ds(k * B + b0, 16)] = wzy * wxs
                        k += 1
            return c2

        lax.fori_loop(0, G, grp_body, 0, unroll=False)

    def fire_gathers(buf):
        for j in range(NSTREAM):
            pltpu.async_copy(
                table_hbm.at[idxb[buf].at[pl.ds(j * ILEN, ILEN)]],
                rowsb[buf].at[pl.ds(j * ILEN, ILEN)],
                semb[buf],
            )

    def wait_gathers(buf):
        for j in range(NSTREAM):
            pltpu.make_async_copy(
                table_hbm.at[idxb[buf].at[pl.ds(j * ILEN, ILEN)]],
                rowsb[buf].at[pl.ds(j * ILEN, ILEN)],
                semb[buf],
            ).wait()

    def accum(buf):
        w_v = wb[buf]
        rows_v = rowsb[buf]
        out_v = outb[buf]

        def acc_body(g, c2):
            b0 = g * 16
            wks = [w_v[pl.ds(k * B + b0, 16)] for k in range(8)]
            for j in range(16):
                lane_j = jnp.full((16,), j, jnp.int32)
                acc = None
                for k in range(8):
                    row = rows_v[k * B + b0 + j]
                    wjk = jnp.take(wks[k], lane_j)
                    term = row * wjk
                    acc = term if acc is None else acc + term
                out_v[2 * g + j // 8, pl.ds((j % 8) * C, 16)] = acc
            return c2

        lax.fori_loop(0, G, acc_body, 0, unroll=False)

    def fire_out(blk, buf):
        base = (wid * NPW + blk * B) // 8
        pltpu.async_copy(outb[buf], out_hbm.at[pl.ds(base, B // 8)],
                         osemb[buf])

    def wait_out(buf):
        pltpu.make_async_copy(outb[buf], out_hbm.at[pl.ds(0, B // 8)],
                              osemb[buf]).wait()

    def sub_block(i, buf):
        nbuf = 1 - buf
        wait_coords(nbuf)
        idx_weights(nbuf)
        fire_gathers(nbuf)
        fire_coords(i + 2, buf)
        wait_gathers(buf)

        @pl.when(i >= 2)
        def _():
            wait_out(buf)

        accum(buf)
        fire_out(i, buf)

    fire_coords(jnp.int32(0), 0)
    fire_coords(jnp.int32(1), 1)
    wait_coords(0)
    idx_weights(0)
    fire_gathers(0)

    def body(d, carry):
        sub_block(2 * d, 0)
        sub_block(2 * d + 1, 1)
        return carry

    lax.fori_loop(0, NBLK // 2, body, 0, unroll=False)

    wait_gathers(0)
    wait_coords(1)
    wait_out(0)
    wait_out(1)


@jax.jit
def kernel(coords, features):
    feat3 = features.reshape(C, GR, W)
    table = _transpose_sc(feat3)
    xs = coords[:, 0]
    ys = coords[:, 1]
    zs = coords[:, 2]
    out = _sample_sc(xs, ys, zs, table)
    return out.reshape(N, 1, C)


# speedup vs baseline: 1.4425x; 1.1695x over previous
import functools

import jax
import jax.numpy as jnp
from jax import lax
from jax.experimental import pallas as pl
from jax.experimental.pallas import tpu as pltpu
from jax.experimental.pallas import tpu_sc as plsc

D = H = W = 128
C = 16
N = 524288
V = D * H * W

NC = 2
NS = 16
NW = NC * NS
NPW = N // NW
B = 256
NBLK = NPW // B
G = B // 16
NIDX = 8 * B
ILEN = 128
NGB = N // B
IROWS = NIDX // ILEN
NSTREAM = NIDX // ILEN

GR = V // W
RPW = GR // NW
RCH = 8
QCH = RCH * W
NCH = RPW // RCH

_mesh = plsc.VectorSubcoreMesh(core_axis_name="c", subcore_axis_name="s")


@functools.partial(
    pl.kernel,
    mesh=_mesh,
    compiler_params=pltpu.CompilerParams(use_tc_tiling_on_sc=False),
    out_type=jax.ShapeDtypeStruct((V, C), jnp.float32),
    scratch_types=[
        pltpu.VMEM((C, RCH, W), jnp.float32),
        pltpu.VMEM((C, RCH, W), jnp.float32),
        pltpu.VMEM((QCH, C), jnp.float32),
        pltpu.VMEM((QCH, C), jnp.float32),
        pltpu.SemaphoreType.DMA,
        pltpu.SemaphoreType.DMA,
        pltpu.SemaphoreType.DMA,
        pltpu.SemaphoreType.DMA,
    ],
)
def _transpose_sc(feat_hbm, table_hbm, chan0, chan1, tout0, tout1,
                  isem0, isem1, osem0, osem1):
    wid = lax.axis_index("s") * NC + lax.axis_index("c")
    lanes = lax.iota(jnp.int32, 16)
    chanb = [chan0, chan1]
    toutb = [tout0, tout1]
    isemb = [isem0, isem1]
    osemb = [osem0, osem1]
    perms = {d: lanes ^ d for d in (1, 2, 4, 8)}
    masks = {}
    for d in (1, 2, 4, 8):
        for bit in (0, d):
            masks[(d, bit)] = (lanes & d) == bit

    def fire_in(ch, buf):
        cc = jnp.minimum(ch, NCH - 1)
        r0 = wid * RPW + cc * RCH
        for c in range(C):
            pltpu.async_copy(feat_hbm.at[c, pl.ds(r0, RCH)],
                             chanb[buf].at[c], isemb[buf])

    def wait_in(buf):
        for c in range(C):
            pltpu.make_async_copy(feat_hbm.at[c, pl.ds(0, RCH)],
                                  chanb[buf].at[c], isemb[buf]).wait()

    def fire_out(ch, buf):
        r0 = wid * RPW + ch * RCH
        pltpu.async_copy(toutb[buf], table_hbm.at[pl.ds(r0 * W, QCH)],
                         osemb[buf])

    def wait_out(buf):
        pltpu.make_async_copy(toutb[buf], table_hbm.at[pl.ds(0, QCH)],
                              osemb[buf]).wait()

    def sub_chunk(ch, buf):
        wait_in(buf)
        chan_v = chanb[buf]
        tout_v = toutb[buf]

        @pl.when(ch >= 2)
        def _():
            wait_out(buf)

        def row_body(r, c2):
            for xb in range(8):
                x0 = xb * 16
                regs = [chan_v[c, r, pl.ds(x0, 16)] for c in range(C)]
                for d in (1, 2, 4, 8):
                    regs = [
                        jnp.where(
                            masks[(d, rr & d)],
                            regs[rr],
                            jnp.take(regs[rr ^ d], perms[d]),
                        )
                        for rr in range(C)
                    ]
                lp0 = r * W + x0
                for j in range(16):
                    tout_v[lp0 + j] = regs[j]
            return c2

        lax.fori_loop(0, RCH, row_body, 0, unroll=False)
        fire_out(ch, buf)
        fire_in(ch + 2, buf)

    fire_in(jnp.int32(0), 0)
    fire_in(jnp.int32(1), 1)

    def body(d2, carry):
        sub_chunk(2 * d2, 0)
        sub_chunk(2 * d2 + 1, 1)
        return carry

    lax.fori_loop(0, NCH // 2, body, 0, unroll=False)

    wait_in(0)
    wait_in(1)
    wait_out(0)
    wait_out(1)


@functools.partial(
    pl.kernel,
    mesh=_mesh,
    compiler_params=pltpu.CompilerParams(use_tc_tiling_on_sc=False),
    out_type=jax.ShapeDtypeStruct((N // 8, 8 * C), jnp.float32),
    scratch_types=[
        pltpu.VMEM((B,), jnp.float32),
        pltpu.VMEM((B,), jnp.float32),
        pltpu.VMEM((B,), jnp.float32),
        pltpu.VMEM((B,), jnp.float32),
        pltpu.VMEM((B,), jnp.float32),
        pltpu.VMEM((B,), jnp.float32),
        pltpu.VMEM((NIDX,), jnp.int32),
        pltpu.VMEM((NIDX,), jnp.int32),
        pltpu.VMEM((NIDX,), jnp.float32),
        pltpu.VMEM((NIDX,), jnp.float32),
        pltpu.VMEM((NIDX, C), jnp.float32),
        pltpu.VMEM((NIDX, C), jnp.float32),
        pltpu.VMEM((B // 8, 8 * C), jnp.float32),
        pltpu.VMEM((B // 8, 8 * C), jnp.float32),
        pltpu.SemaphoreType.DMA,
        pltpu.SemaphoreType.DMA,
        pltpu.SemaphoreType.DMA,
        pltpu.SemaphoreType.DMA,
        pltpu.SemaphoreType.DMA,
        pltpu.SemaphoreType.DMA,
    ],
)
def _sample_sc(xs_hbm, ys_hbm, zs_hbm, table_hbm, out_hbm,
               xv0, yv0, zv0, xv1, yv1, zv1,
               idx0, idx1, w0, w1, rows0, rows1, out0, out1,
               sem0, sem1, csem0, csem1, osem0, osem1):
    wid = lax.axis_index("s") * NC + lax.axis_index("c")
    lanes = lax.iota(jnp.int32, 16)
    cv = [(xv0, yv0, zv0), (xv1, yv1, zv1)]
    idxb = [idx0, idx1]
    wb = [w0, w1]
    rowsb = [rows0, rows1]
    outb = [out0, out1]
    semb = [sem0, sem1]
    csemb = [csem0, csem1]
    osemb = [osem0, osem1]

    def fire_coords(blk, buf):
        bc = jnp.minimum(blk, NBLK - 1)
        base = wid * NPW + bc * B
        pltpu.async_copy(xs_hbm.at[pl.ds(base, B)], cv[buf][0], csemb[buf])
        pltpu.async_copy(ys_hbm.at[pl.ds(base, B)], cv[buf][1], csemb[buf])
        pltpu.async_copy(zs_hbm.at[pl.ds(base, B)], cv[buf][2], csemb[buf])

    def wait_coords(buf):
        for r in cv[buf]:
            pltpu.make_async_copy(xs_hbm.at[pl.ds(0, B)], r, csemb[buf]).wait()

    def idx_weights(buf):
        xv, yv, zv = cv[buf]
        idx_v = idxb[buf]
        w_v = wb[buf]

        def grp_body(g, c2):
            b0 = g * 16
            cx = xv[pl.ds(b0, 16)]
            cy = yv[pl.ds(b0, 16)]
            cz = zv[pl.ds(b0, 16)]

            def axis(cu, ext):
                gg = cu * 2.0 - 1.0
                u = (gg + 1.0) * 0.5 * (ext - 1)
                u = jnp.minimum(jnp.maximum(u, 0.0), float(ext - 1))
                u0 = u.astype(jnp.int32)
                wu = u - u0.astype(jnp.float32)
                u1 = jnp.minimum(u0 + 1, ext - 1)
                return u0, u1, wu

            x0, x1, wx = axis(cx, W)
            y0, y1, wy = axis(cy, H)
            z0, z1, wz = axis(cz, D)
            wx0 = 1.0 - wx
            wy0 = 1.0 - wy
            wz0 = 1.0 - wz
            k = 0
            for dz in (0, 1):
                zi = z1 if dz else z0
                wzs = wz if dz else wz0
                for dy in (0, 1):
                    yi = y1 if dy else y0
                    wys = wy if dy else wy0
                    zy = (zi * H + yi) * W
                    wzy = wzs * wys
                    for dx in (0, 1):
                        xi = x1 if dx else x0
                        wxs = wx if dx else wx0
                        idx_v[pl.ds(k * B + b0, 16)] = zy + xi
                        w_v[pl.ds(k * B + b0, 16)] = wzy * wxs
                        k += 1
            return c2

        lax.fori_loop(0, G, grp_body, 0, unroll=False)

    def fire_gathers(buf):
        for j in range(NSTREAM):
            pltpu.async_copy(
                table_hbm.at[idxb[buf].at[pl.ds(j * ILEN, ILEN)]],
                rowsb[buf].at[pl.ds(j * ILEN, ILEN)],
                semb[buf],
            )

    def wait_gathers(buf):
        for j in range(NSTREAM):
            pltpu.make_async_copy(
                table_hbm.at[idxb[buf].at[pl.ds(j * ILEN, ILEN)]],
                rowsb[buf].at[pl.ds(j * ILEN, ILEN)],
                semb[buf],
            ).wait()

    def accum(buf):
        w_v = wb[buf]
        rows_v = rowsb[buf]
        out_v = outb[buf]

        def acc_body(g, c2):
            b0 = g * 16
            wks = [w_v[pl.ds(k * B + b0, 16)] for k in range(8)]
            for j in range(16):
                lane_j = jnp.full((16,), j, jnp.int32)
                acc = None
                for k in range(8):
                    row = rows_v[k * B + b0 + j]
                    wjk = jnp.take(wks[k], lane_j)
                    term = row * wjk
                    acc = term if acc is None else acc + term
                out_v[2 * g + j // 8, pl.ds((j % 8) * C, 16)] = acc
            return c2

        lax.fori_loop(0, G, acc_body, 0, unroll=False)

    def fire_out(blk, buf):
        base = (wid * NPW + blk * B) // 8
        pltpu.async_copy(outb[buf], out_hbm.at[pl.ds(base, B // 8)],
                         osemb[buf])

    def wait_out(buf):
        pltpu.make_async_copy(outb[buf], out_hbm.at[pl.ds(0, B // 8)],
                              osemb[buf]).wait()

    def sub_block(i, buf):
        nbuf = 1 - buf
        wait_coords(nbuf)
        idx_weights(nbuf)
        fire_gathers(nbuf)
        fire_coords(i + 2, buf)
        wait_gathers(buf)

        @pl.when(i >= 2)
        def _():
            wait_out(buf)

        accum(buf)
        fire_out(i, buf)

    fire_coords(jnp.int32(0), 0)
    fire_coords(jnp.int32(1), 1)
    wait_coords(0)
    idx_weights(0)
    fire_gathers(0)

    def body(d, carry):
        sub_block(2 * d, 0)
        sub_block(2 * d + 1, 1)
        return carry

    lax.fori_loop(0, NBLK // 2, body, 0, unroll=False)

    wait_gathers(0)
    wait_coords(1)
    wait_out(0)
    wait_out(1)


@jax.jit
def kernel(coords, features):
    feat3 = features.reshape(C, GR, W)
    table = _transpose_sc(feat3)
    xs = coords[:, 0]
    ys = coords[:, 1]
    zs = coords[:, 2]
    out = _sample_sc(xs, ys, zs, table)
    return out.reshape(N, 1, C)


# speedup vs baseline: 1.4505x; 1.0056x over previous
import functools

import jax
import jax.numpy as jnp
from jax import lax
from jax.experimental import pallas as pl
from jax.experimental.pallas import tpu as pltpu
from jax.experimental.pallas import tpu_sc as plsc

D = H = W = 128
C = 16
N = 524288
V = D * H * W

NC = 2
NS = 16
NW = NC * NS
NPW = N // NW
B = 256
NBLK = NPW // B
G = B // 16
NIDX = 8 * B
ILEN = 128
NGB = N // B
IROWS = NIDX // ILEN
NSTREAM = NIDX // ILEN

GR = V // W
RPW = GR // NW
RCH = 8
QCH = RCH * W
NCH = RPW // RCH

_mesh = plsc.VectorSubcoreMesh(core_axis_name="c", subcore_axis_name="s")


@functools.partial(
    pl.kernel,
    mesh=_mesh,
    compiler_params=pltpu.CompilerParams(use_tc_tiling_on_sc=False),
    out_type=jax.ShapeDtypeStruct((V, C), jnp.float32),
    scratch_types=[
        pltpu.VMEM((C, RCH, W), jnp.float32),
        pltpu.VMEM((C, RCH, W), jnp.float32),
        pltpu.VMEM((QCH, C), jnp.float32),
        pltpu.VMEM((QCH, C), jnp.float32),
        pltpu.SemaphoreType.DMA,
        pltpu.SemaphoreType.DMA,
        pltpu.SemaphoreType.DMA,
        pltpu.SemaphoreType.DMA,
    ],
)
def _transpose_sc(feat_hbm, table_hbm, chan0, chan1, tout0, tout1,
                  isem0, isem1, osem0, osem1):
    wid = lax.axis_index("s") * NC + lax.axis_index("c")
    lanes = lax.iota(jnp.int32, 16)
    chanb = [chan0, chan1]
    toutb = [tout0, tout1]
    isemb = [isem0, isem1]
    osemb = [osem0, osem1]
    perms = {d: lanes ^ d for d in (1, 2, 4, 8)}
    masks = {}
    for d in (1, 2, 4, 8):
        for bit in (0, d):
            masks[(d, bit)] = (lanes & d) == bit

    def fire_in(ch, buf):
        cc = jnp.minimum(ch, NCH - 1)
        r0 = wid * RPW + cc * RCH
        for c in range(C):
            pltpu.async_copy(feat_hbm.at[c, pl.ds(r0, RCH)],
                             chanb[buf].at[c], isemb[buf])

    def wait_in(buf):
        pltpu.make_async_copy(feat_hbm.at[pl.ds(0, C), pl.ds(0, RCH)],
                              chanb[buf], isemb[buf]).wait()

    def fire_out(ch, buf):
        r0 = wid * RPW + ch * RCH
        pltpu.async_copy(toutb[buf], table_hbm.at[pl.ds(r0 * W, QCH)],
                         osemb[buf])

    def wait_out(buf):
        pltpu.make_async_copy(toutb[buf], table_hbm.at[pl.ds(0, QCH)],
                              osemb[buf]).wait()

    def sub_chunk(ch, buf):
        wait_in(buf)
        chan_v = chanb[buf]
        tout_v = toutb[buf]

        @pl.when(ch >= 2)
        def _():
            wait_out(buf)

        def row_body(r, c2):
            for xb in range(8):
                x0 = xb * 16
                regs = [chan_v[c, r, pl.ds(x0, 16)] for c in range(C)]
                for d in (1, 2, 4, 8):
                    regs = [
                        jnp.where(
                            masks[(d, rr & d)],
                            regs[rr],
                            jnp.take(regs[rr ^ d], perms[d]),
                        )
                        for rr in range(C)
                    ]
                lp0 = r * W + x0
                for j in range(16):
                    tout_v[lp0 + j] = regs[j]
            return c2

        lax.fori_loop(0, RCH, row_body, 0, unroll=False)
        fire_out(ch, buf)
        fire_in(ch + 2, buf)

    fire_in(jnp.int32(0), 0)
    fire_in(jnp.int32(1), 1)

    def body(d2, carry):
        sub_chunk(2 * d2, 0)
        sub_chunk(2 * d2 + 1, 1)
        return carry

    lax.fori_loop(0, NCH // 2, body, 0, unroll=False)

    wait_in(0)
    wait_in(1)
    wait_out(0)
    wait_out(1)


@functools.partial(
    pl.kernel,
    mesh=_mesh,
    compiler_params=pltpu.CompilerParams(use_tc_tiling_on_sc=False),
    out_type=jax.ShapeDtypeStruct((N // 8, 8 * C), jnp.float32),
    scratch_types=[
        pltpu.VMEM((B,), jnp.float32),
        pltpu.VMEM((B,), jnp.float32),
        pltpu.VMEM((B,), jnp.float32),
        pltpu.VMEM((B,), jnp.float32),
        pltpu.VMEM((B,), jnp.float32),
        pltpu.VMEM((B,), jnp.float32),
        pltpu.VMEM((NIDX,), jnp.int32),
        pltpu.VMEM((NIDX,), jnp.int32),
        pltpu.VMEM((NIDX,), jnp.float32),
        pltpu.VMEM((NIDX,), jnp.float32),
        pltpu.VMEM((NIDX, C), jnp.float32),
        pltpu.VMEM((NIDX, C), jnp.float32),
        pltpu.VMEM((B // 8, 8 * C), jnp.float32),
        pltpu.VMEM((B // 8, 8 * C), jnp.float32),
        pltpu.SemaphoreType.DMA,
        pltpu.SemaphoreType.DMA,
        pltpu.SemaphoreType.DMA,
        pltpu.SemaphoreType.DMA,
        pltpu.SemaphoreType.DMA,
        pltpu.SemaphoreType.DMA,
    ],
)
def _sample_sc(xs_hbm, ys_hbm, zs_hbm, table_hbm, out_hbm,
               xv0, yv0, zv0, xv1, yv1, zv1,
               idx0, idx1, w0, w1, rows0, rows1, out0, out1,
               sem0, sem1, csem0, csem1, osem0, osem1):
    wid = lax.axis_index("s") * NC + lax.axis_index("c")
    lanes = lax.iota(jnp.int32, 16)
    cv = [(xv0, yv0, zv0), (xv1, yv1, zv1)]
    idxb = [idx0, idx1]
    wb = [w0, w1]
    rowsb = [rows0, rows1]
    outb = [out0, out1]
    semb = [sem0, sem1]
    csemb = [csem0, csem1]
    osemb = [osem0, osem1]

    def fire_coords(blk, buf):
        bc = jnp.minimum(blk, NBLK - 1)
        base = wid * NPW + bc * B
        pltpu.async_copy(xs_hbm.at[pl.ds(base, B)], cv[buf][0], csemb[buf])
        pltpu.async_copy(ys_hbm.at[pl.ds(base, B)], cv[buf][1], csemb[buf])
        pltpu.async_copy(zs_hbm.at[pl.ds(base, B)], cv[buf][2], csemb[buf])

    def wait_coords(buf):
        for r in cv[buf]:
            pltpu.make_async_copy(xs_hbm.at[pl.ds(0, B)], r, csemb[buf]).wait()

    def idx_weights(buf):
        xv, yv, zv = cv[buf]
        idx_v = idxb[buf]
        w_v = wb[buf]

        def grp_body(g, c2):
            b0 = g * 16
            cx = xv[pl.ds(b0, 16)]
            cy = yv[pl.ds(b0, 16)]
            cz = zv[pl.ds(b0, 16)]

            def axis(cu, ext):
                gg = cu * 2.0 - 1.0
                u = (gg + 1.0) * 0.5 * (ext - 1)
                u = jnp.minimum(jnp.maximum(u, 0.0), float(ext - 1))
                u0 = u.astype(jnp.int32)
                wu = u - u0.astype(jnp.float32)
                u1 = jnp.minimum(u0 + 1, ext - 1)
                return u0, u1, wu

            x0, x1, wx = axis(cx, W)
            y0, y1, wy = axis(cy, H)
            z0, z1, wz = axis(cz, D)
            wx0 = 1.0 - wx
            wy0 = 1.0 - wy
            wz0 = 1.0 - wz
            k = 0
            for dz in (0, 1):
                zi = z1 if dz else z0
                wzs = wz if dz else wz0
                for dy in (0, 1):
                    yi = y1 if dy else y0
                    wys = wy if dy else wy0
                    zy = (zi * H + yi) * W
                    wzy = wzs * wys
                    for dx in (0, 1):
                        xi = x1 if dx else x0
                        wxs = wx if dx else wx0
                        idx_v[pl.ds(k * B + b0, 16)] = zy + xi
                        w_v[pl.ds(k * B + b0, 16)] = wzy * wxs
                        k += 1
            return c2

        lax.fori_loop(0, G, grp_body, 0, unroll=False)

    def fire_gathers(buf):
        for j in range(NSTREAM):
            pltpu.async_copy(
                table_hbm.at[idxb[buf].at[pl.ds(j * ILEN, ILEN)]],
                rowsb[buf].at[pl.ds(j * ILEN, ILEN)],
                semb[buf],
            )

    def wait_gathers(buf):
        pltpu.make_async_copy(
            table_hbm.at[idxb[buf]], rowsb[buf], semb[buf]
        ).wait()

    def accum(buf):
        w_v = wb[buf]
        rows_v = rowsb[buf]
        out_v = outb[buf]

        def acc_body(g, c2):
            b0 = g * 16
            wks = [w_v[pl.ds(k * B + b0, 16)] for k in range(8)]
            for j in range(16):
                lane_j = jnp.full((16,), j, jnp.int32)
                acc = None
                for k in range(8):
                    row = rows_v[k * B + b0 + j]
                    wjk = jnp.take(wks[k], lane_j)
                    term = row * wjk
                    acc = term if acc is None else acc + term
                out_v[2 * g + j // 8, pl.ds((j % 8) * C, 16)] = acc
            return c2

        lax.fori_loop(0, G, acc_body, 0, unroll=False)

    def fire_out(blk, buf):
        base = (wid * NPW + blk * B) // 8
        pltpu.async_copy(outb[buf], out_hbm.at[pl.ds(base, B // 8)],
                         osemb[buf])

    def wait_out(buf):
        pltpu.make_async_copy(outb[buf], out_hbm.at[pl.ds(0, B // 8)],
                              osemb[buf]).wait()

    def sub_block(i, buf):
        nbuf = 1 - buf
        wait_coords(nbuf)
        idx_weights(nbuf)
        fire_gathers(nbuf)
        fire_coords(i + 2, buf)
        wait_gathers(buf)

        @pl.when(i >= 2)
        def _():
            wait_out(buf)

        accum(buf)
        fire_out(i, buf)

    fire_coords(jnp.int32(0), 0)
    fire_coords(jnp.int32(1), 1)
    wait_coords(0)
    idx_weights(0)
    fire_gathers(0)

    def body(d, carry):
        sub_block(2 * d, 0)
        sub_block(2 * d + 1, 1)
        return carry

    lax.fori_loop(0, NBLK // 2, body, 0, unroll=False)

    wait_gathers(0)
    wait_coords(1)
    wait_out(0)
    wait_out(1)


@jax.jit
def kernel(coords, features):
    feat3 = features.reshape(C, GR, W)
    table = _transpose_sc(feat3)
    xs = coords[:, 0]
    ys = coords[:, 1]
    zs = coords[:, 2]
    out = _sample_sc(xs, ys, zs, table)
    return out.reshape(N, 1, C)


# speedup vs baseline: 1.4517x; 1.0008x over previous
import functools

import jax
import jax.numpy as jnp
from jax import lax
from jax.experimental import pallas as pl
from jax.experimental.pallas import tpu as pltpu
from jax.experimental.pallas import tpu_sc as plsc

D = H = W = 128
C = 16
N = 524288
V = D * H * W

NC = 2
NS = 16
NW = NC * NS
NPW = N // NW
B = 256
NBLK = NPW // B
G = B // 16
NIDX = 8 * B
ILEN = 128
NSTREAM = NIDX // ILEN

GR = V // W
RPW = GR // NW
RCH = 8
QCH = RCH * W
NCH = RPW // RCH

_mesh = plsc.VectorSubcoreMesh(core_axis_name="c", subcore_axis_name="s")


@functools.partial(
    pl.kernel,
    mesh=_mesh,
    compiler_params=pltpu.CompilerParams(use_tc_tiling_on_sc=False),
    out_type=jax.ShapeDtypeStruct((V, C), jnp.float32),
    scratch_types=[
        pltpu.VMEM((C, RCH, W), jnp.float32),
        pltpu.VMEM((C, RCH, W), jnp.float32),
        pltpu.VMEM((QCH, C), jnp.float32),
        pltpu.VMEM((QCH, C), jnp.float32),
        pltpu.SemaphoreType.DMA,
        pltpu.SemaphoreType.DMA,
        pltpu.SemaphoreType.DMA,
        pltpu.SemaphoreType.DMA,
    ],
)
def _transpose_sc(feat_hbm, table_hbm, chan0, chan1, tout0, tout1,
                  isem0, isem1, osem0, osem1):
    wid = lax.axis_index("s") * NC + lax.axis_index("c")
    lanes = lax.iota(jnp.int32, 16)
    chanb = [chan0, chan1]
    toutb = [tout0, tout1]
    isemb = [isem0, isem1]
    osemb = [osem0, osem1]
    perms = {d: lanes ^ d for d in (1, 2, 4, 8)}
    masks = {}
    for d in (1, 2, 4, 8):
        for bit in (0, d):
            masks[(d, bit)] = (lanes & d) == bit

    def fire_in(ch, buf):
        cc = jnp.minimum(ch, NCH - 1)
        r0 = wid * RPW + cc * RCH
        for c in range(C):
            pltpu.async_copy(feat_hbm.at[c, pl.ds(r0, RCH)],
                             chanb[buf].at[c], isemb[buf])

    def wait_in(buf):
        pltpu.make_async_copy(feat_hbm.at[pl.ds(0, C), pl.ds(0, RCH)],
                              chanb[buf], isemb[buf]).wait()

    def fire_out(ch, buf):
        r0 = wid * RPW + ch * RCH
        pltpu.async_copy(toutb[buf], table_hbm.at[pl.ds(r0 * W, QCH)],
                         osemb[buf])

    def wait_out(buf):
        pltpu.make_async_copy(toutb[buf], table_hbm.at[pl.ds(0, QCH)],
                              osemb[buf]).wait()

    def sub_chunk(ch, buf):
        wait_in(buf)
        chan_v = chanb[buf]
        tout_v = toutb[buf]

        @pl.when(ch >= 2)
        def _():
            wait_out(buf)

        def row_body(r, c2):
            for xb in range(8):
                x0 = xb * 16
                regs = [chan_v[c, r, pl.ds(x0, 16)] for c in range(C)]
                for d in (1, 2, 4, 8):
                    regs = [
                        jnp.where(
                            masks[(d, rr & d)],
                            regs[rr],
                            jnp.take(regs[rr ^ d], perms[d]),
                        )
                        for rr in range(C)
                    ]
                lp0 = r * W + x0
                for j in range(16):
                    tout_v[lp0 + j] = regs[j]
            return c2

        lax.fori_loop(0, RCH, row_body, 0, unroll=False)
        fire_out(ch, buf)
        fire_in(ch + 2, buf)

    fire_in(jnp.int32(0), 0)
    fire_in(jnp.int32(1), 1)

    def body(d2, carry):
        sub_chunk(2 * d2, 0)
        sub_chunk(2 * d2 + 1, 1)
        return carry

    lax.fori_loop(0, NCH // 2, body, 0, unroll=False)

    wait_in(0)
    wait_in(1)
    wait_out(0)
    wait_out(1)


@functools.partial(
    pl.kernel,
    mesh=_mesh,
    compiler_params=pltpu.CompilerParams(use_tc_tiling_on_sc=False),
    out_type=jax.ShapeDtypeStruct((N // 8, 8 * C), jnp.float32),
    scratch_types=[
        pltpu.VMEM((B,), jnp.float32),
        pltpu.VMEM((B,), jnp.float32),
        pltpu.VMEM((B,), jnp.float32),
        pltpu.VMEM((B,), jnp.float32),
        pltpu.VMEM((B,), jnp.float32),
        pltpu.VMEM((B,), jnp.float32),
        pltpu.VMEM((NIDX,), jnp.int32),
        pltpu.VMEM((NIDX,), jnp.int32),
        pltpu.VMEM((NIDX,), jnp.float32),
        pltpu.VMEM((NIDX,), jnp.float32),
        pltpu.VMEM((NIDX, C), jnp.float32),
        pltpu.VMEM((NIDX, C), jnp.float32),
        pltpu.VMEM((B // 8, 8 * C), jnp.float32),
        pltpu.VMEM((B // 8, 8 * C), jnp.float32),
        pltpu.SemaphoreType.DMA,
        pltpu.SemaphoreType.DMA,
        pltpu.SemaphoreType.DMA,
        pltpu.SemaphoreType.DMA,
        pltpu.SemaphoreType.DMA,
        pltpu.SemaphoreType.DMA,
    ],
)
def _sample_sc(xs_hbm, ys_hbm, zs_hbm, table_hbm, out_hbm,
               xv0, yv0, zv0, xv1, yv1, zv1,
               idx0, idx1, w0, w1, rows0, rows1, out0, out1,
               sem0, sem1, csem0, csem1, osem0, osem1):
    wid = lax.axis_index("s") * NC + lax.axis_index("c")
    lanes = lax.iota(jnp.int32, 16)
    cv = [(xv0, yv0, zv0), (xv1, yv1, zv1)]
    idxb = [idx0, idx1]
    wb = [w0, w1]
    rowsb = [rows0, rows1]
    outb = [out0, out1]
    semb = [sem0, sem1]
    csemb = [csem0, csem1]
    osemb = [osem0, osem1]

    def fire_coords(blk, buf):
        bc = jnp.minimum(blk, NBLK - 1)
        base = wid * NPW + bc * B
        pltpu.async_copy(xs_hbm.at[pl.ds(base, B)], cv[buf][0], csemb[buf])
        pltpu.async_copy(ys_hbm.at[pl.ds(base, B)], cv[buf][1], csemb[buf])
        pltpu.async_copy(zs_hbm.at[pl.ds(base, B)], cv[buf][2], csemb[buf])

    def wait_coords(buf):
        for r in cv[buf]:
            pltpu.make_async_copy(xs_hbm.at[pl.ds(0, B)], r, csemb[buf]).wait()

    def idx_weights(buf):
        xv, yv, zv = cv[buf]
        idx_v = idxb[buf]
        w_v = wb[buf]

        def grp_body(g, c2):
            b0 = g * 16
            cx = xv[pl.ds(b0, 16)]
            cy = yv[pl.ds(b0, 16)]
            cz = zv[pl.ds(b0, 16)]

            def axis(cu, ext):
                gg = cu * 2.0 - 1.0
                u = (gg + 1.0) * 0.5 * (ext - 1)
                u = jnp.minimum(jnp.maximum(u, 0.0), float(ext - 1))
                u0 = u.astype(jnp.int32)
                wu = u - u0.astype(jnp.float32)
                u1 = jnp.minimum(u0 + 1, ext - 1)
                return u0, u1, wu

            x0, x1, wx = axis(cx, W)
            y0, y1, wy = axis(cy, H)
            z0, z1, wz = axis(cz, D)
            wx0 = 1.0 - wx
            wy0 = 1.0 - wy
            wz0 = 1.0 - wz
            k = 0
            for dz in (0, 1):
                zi = z1 if dz else z0
                wzs = wz if dz else wz0
                for dy in (0, 1):
                    yi = y1 if dy else y0
                    wys = wy if dy else wy0
                    zy = (zi * H + yi) * W
                    wzy = wzs * wys
                    for dx in (0, 1):
                        xi = x1 if dx else x0
                        wxs = wx if dx else wx0
                        idx_v[pl.ds(k * B + b0, 16)] = zy + xi
                        w_v[pl.ds(k * B + b0, 16)] = wzy * wxs
                        k += 1
            return c2

        lax.fori_loop(0, G, grp_body, 0, unroll=False)

    def fire_gathers(buf):
        for j in range(NSTREAM):
            pltpu.async_copy(
                table_hbm.at[idxb[buf].at[pl.ds(j * ILEN, ILEN)]],
                rowsb[buf].at[pl.ds(j * ILEN, ILEN)],
                semb[buf],
            )

    def wait_gathers(buf):
        pltpu.make_async_copy(
            table_hbm.at[idxb[buf]], rowsb[buf], semb[buf]
        ).wait()

    def accum(buf):
        w_v = wb[buf]
        rows_v = rowsb[buf]
        out_v = outb[buf]

        def acc_body(g, c2):
            b0 = g * 16
            wks = [w_v[pl.ds(k * B + b0, 16)] for k in range(8)]
            for j in range(16):
                lane_j = jnp.full((16,), j, jnp.int32)
                acc = None
                for k in range(8):
                    row = rows_v[k * B + b0 + j]
                    wjk = jnp.take(wks[k], lane_j)
                    term = row * wjk
                    acc = term if acc is None else acc + term
                out_v[2 * g + j // 8, pl.ds((j % 8) * C, 16)] = acc
            return c2

        lax.fori_loop(0, G, acc_body, 0, unroll=False)

    def fire_out(blk, buf):
        base = (wid * NPW + blk * B) // 8
        pltpu.async_copy(outb[buf], out_hbm.at[pl.ds(base, B // 8)],
                         osemb[buf])

    def wait_out(buf):
        pltpu.make_async_copy(outb[buf], out_hbm.at[pl.ds(0, B // 8)],
                              osemb[buf]).wait()

    def sub_block(i, buf):
        nbuf = 1 - buf
        wait_coords(nbuf)
        idx_weights(nbuf)
        fire_gathers(nbuf)
        fire_coords(i + 2, buf)
        wait_gathers(buf)

        @pl.when(i >= 2)
        def _():
            wait_out(buf)

        accum(buf)
        fire_out(i, buf)

    fire_coords(jnp.int32(0), 0)
    fire_coords(jnp.int32(1), 1)
    wait_coords(0)
    idx_weights(0)
    fire_gathers(0)

    def body(d, carry):
        sub_block(2 * d, 0)
        sub_block(2 * d + 1, 1)
        return carry

    lax.fori_loop(0, NBLK // 2, body, 0, unroll=False)

    wait_gathers(0)
    wait_coords(1)
    wait_out(0)
    wait_out(1)


@jax.jit
def kernel(coords, features):
    feat3 = features.reshape(C, GR, W)
    table = _transpose_sc(feat3)
    xs = coords[:, 0]
    ys = coords[:, 1]
    zs = coords[:, 2]
    out = _sample_sc(xs, ys, zs, table)
    return out.reshape(N, 1, C)
